# baseline (speedup 1.0000x reference)
import functools

import jax
import jax.numpy as jnp
from jax import lax
from jax.experimental import pallas as pl
from jax.experimental.pallas import tpu as pltpu
from jax.experimental.pallas import tpu_sc as plsc

N, K, D, E = 10000, 32, 128, 4096
NC, NS = 2, 16
NT = NC * NS
NPT = 322
NPAD = NT * NPT
CN = 2
ROWS = CN * K
NCHUNK = NPT // CN
EPT = E // NS

_mesh = plsc.VectorSubcoreMesh(core_axis_name="c", subcore_axis_name="s")


@functools.partial(
    pl.kernel,
    mesh=_mesh,
    out_type=jax.ShapeDtypeStruct((NC, E, D), jnp.float32),
    scratch_types=[
        pltpu.VMEM((2 * CN, D), jnp.float32),
        pltpu.VMEM((NCHUNK, ROWS), jnp.int32),
        pltpu.VMEM((NPT, K), jnp.float32),
        pltpu.VMEM((2 * ROWS, D), jnp.float32),
        pltpu.VMEM_SHARED((E, D), jnp.float32),
        pltpu.SemaphoreType.DMA,
        pltpu.SemaphoreType.DMA,
    ],
)
def _scatter_kernel(x_hbm, idx_hbm, w_hbm, z_hbm, out_hbm,
                    xc_v, idx_v, w_v, contrib_v, he_sh, semx, sems):
    c = lax.axis_index("c")
    s = lax.axis_index("s")
    t = c * NS + s
    pltpu.sync_copy(idx_hbm.at[t], idx_v)
    pltpu.sync_copy(w_hbm.at[t], w_v)
    pltpu.sync_copy(z_hbm.at[pl.ds(s * EPT, EPT)],
                    he_sh.at[pl.ds(s * EPT, EPT)])
    plsc.subcore_barrier()

    def x_copy(j, p):
        return pltpu.make_async_copy(
            x_hbm.at[t, pl.ds(j * CN, CN)], xc_v.at[pl.ds(p * CN, CN)], semx)

    def scat_copy(j, p):
        return pltpu.make_async_copy(
            contrib_v.at[pl.ds(p * ROWS, ROWS)], he_sh.at[idx_v.at[j]], sems)

    def compute(j, p):
        for i in range(CN):
            node = j * CN + i
            wvecs = [w_v[node, pl.ds(h * 16, 16)] for h in range(K // 16)]
            ws = [wvecs[k // 16][k % 16] for k in range(K)]
            for col in range(D // 16):
                vx = xc_v[p * CN + i, pl.ds(col * 16, 16)]
                for k in range(K):
                    contrib_v[p * ROWS + i * K + k,
                              pl.ds(col * 16, 16)] = vx * ws[k]

    def step(j, p):
        x_copy(j, p).wait()
        x_copy(jnp.minimum(j + 1, NCHUNK - 1), 1 - p).start()
        compute(j, p)
        scat_copy(j - 1, 1 - p).wait()
        scat_copy(j, p).start(add=True)

    x_copy(0, 0).start()
    x_copy(0, 0).wait()
    x_copy(1, 1).start()
    compute(0, 0)
    scat_copy(0, 0).start(add=True)

    def pair(m, carry):
        step(2 * m + 1, 1)
        step(2 * m + 2, 0)
        return carry

    lax.fori_loop(0, (NCHUNK - 1) // 2, pair, 0)
    scat_copy(NCHUNK - 1, 0).wait()
    x_copy(NCHUNK - 1, 1).wait()
    plsc.subcore_barrier()
    pltpu.sync_copy(he_sh.at[pl.ds(s * EPT, EPT)],
                    out_hbm.at[c, pl.ds(s * EPT, EPT)])


def _sc_gelu(z):
    u = z * 0.7071067811865476
    au = jnp.abs(u)
    t = 1.0 / (1.0 + 0.3275911 * au)
    poly = ((((1.061405429 * t - 1.453152027) * t + 1.421413741) * t
             - 0.284496736) * t + 0.254829592) * t
    erf_abs = 1.0 - poly * jnp.exp(-(au * au))
    erf_u = jnp.where(u < 0.0, -erf_abs, erf_abs)
    return z * 0.5 * (1.0 + erf_u)


def _sc_rsqrt(v16):
    i = lax.bitcast_convert_type(v16, jnp.int32)
    i = 0x5F3759DF - lax.shift_right_logical(i, 1)
    y = lax.bitcast_convert_type(i, jnp.float32)
    for _ in range(3):
        y = y * (1.5 - 0.5 * v16 * y * y)
    return y


def _lane_sum(x16):
    for d in (1, 2, 4, 8):
        idx = lax.iota(jnp.int32, 16) ^ d
        x16 = x16 + lax.gather(
            x16, idx[:, None],
            lax.GatherDimensionNumbers(offset_dims=(),
                                       collapsed_slice_dims=(0,),
                                       start_index_map=(0,)),
            (1,), mode=lax.GatherScatterMode.PROMISE_IN_BOUNDS)
    return x16


@functools.partial(
    pl.kernel,
    mesh=_mesh,
    out_type=jax.ShapeDtypeStruct((NT, NPT, D), jnp.float32),
    scratch_types=[
        pltpu.VMEM((NCHUNK, ROWS), jnp.int32),
        pltpu.VMEM((NPT, K), jnp.float32),
        pltpu.VMEM((2 * ROWS, D), jnp.float32),
        pltpu.VMEM((2 * CN, D), jnp.float32),
        pltpu.VMEM((2 * CN, D), jnp.float32),
        pltpu.VMEM((3, D), jnp.float32),
        pltpu.VMEM_SHARED((E, D), jnp.float32),
        pltpu.SemaphoreType.DMA,
        pltpu.SemaphoreType.DMA,
        pltpu.SemaphoreType.DMA,
    ],
)
def _gather_kernel(g_hbm, idx_hbm, w_hbm, x_hbm, prm_hbm, out_hbm,
                   idx_v, w_v, rows_v, oc_v, xr_v, p_v, he_sh,
                   semg, semo, semx):
    c = lax.axis_index("c")
    s = lax.axis_index("s")
    t = c * NS + s
    pltpu.sync_copy(idx_hbm.at[t], idx_v)
    pltpu.sync_copy(w_hbm.at[t], w_v)
    pltpu.sync_copy(prm_hbm, p_v)
    pltpu.sync_copy(g_hbm.at[pl.ds(s * EPT, EPT)],
                    he_sh.at[pl.ds(s * EPT, EPT)])
    plsc.subcore_barrier()

    def g_copy(j, p):
        return pltpu.make_async_copy(
            he_sh.at[idx_v.at[j]], rows_v.at[pl.ds(p * ROWS, ROWS)], semg)

    def o_copy(j, p):
        return pltpu.make_async_copy(
            oc_v.at[pl.ds(p * CN, CN)], out_hbm.at[t, pl.ds(j * CN, CN)],
            semo)

    def x_copy(j, p):
        return pltpu.make_async_copy(
            x_hbm.at[t, pl.ds(j * CN, CN)], xr_v.at[pl.ds(p * CN, CN)], semx)

    def compute(j, p):
        for i in range(CN):
            node = j * CN + i
            wvecs = [w_v[node, pl.ds(h * 16, 16)] for h in range(K // 16)]
            ws = [wvecs[k // 16][k % 16] for k in range(K)]
            base = p * ROWS + i * K
            a = []
            for col in range(D // 16):
                cs = pl.ds(col * 16, 16)
                acc = rows_v[base, cs] * ws[0]
                for k in range(1, K):
                    acc = acc + rows_v[base + k, cs] * ws[k]
                a.append(_sc_gelu(acc + p_v[0, cs]))
            ssum = a[0]
            for col in range(1, D // 16):
                ssum = ssum + a[col]
            mu = _lane_sum(ssum) * (1.0 / D)
            d = [av - mu for av in a]
            qsum = d[0] * d[0]
            for col in range(1, D // 16):
                qsum = qsum + d[col] * d[col]
            var = _lane_sum(qsum) * (1.0 / D)
            inv = _sc_rsqrt(var + 1e-5)
            for col in range(D // 16):
                cs = pl.ds(col * 16, 16)
                oc_v[p * CN + i, cs] = (d[col] * inv * p_v[1, cs]
                                        + p_v[2, cs] + xr_v[p * CN + i, cs])

    def step(j, p):
        g_copy(j, p).wait()
        g_copy(jnp.minimum(j + 1, NCHUNK - 1), 1 - p).start()
        x_copy(j, p).wait()
        x_copy(jnp.minimum(j + 1, NCHUNK - 1), 1 - p).start()
        compute(j, p)
        o_copy(j - 1, 1 - p).wait()
        o_copy(j, p).start()

    g_copy(0, 0).start()
    x_copy(0, 0).start()
    g_copy(0, 0).wait()
    x_copy(0, 0).wait()
    g_copy(1, 1).start()
    x_copy(1, 1).start()
    compute(0, 0)
    o_copy(0, 0).start()

    def pair(m, carry):
        step(2 * m + 1, 1)
        step(2 * m + 2, 0)
        return carry

    lax.fori_loop(0, (NCHUNK - 1) // 2, pair, 0)
    o_copy(NCHUNK - 1, 0).wait()
    g_copy(NCHUNK - 1, 1).wait()
    x_copy(NCHUNK - 1, 1).wait()


def _edge_mlp_body(hp_ref, w_ref, b_ref, g_ref, beta_ref, w2t_ref, out_ref):
    h = hp_ref[0] + hp_ref[1]
    z = jnp.dot(h, w_ref[...], preferred_element_type=jnp.float32)
    z = z + b_ref[...]
    a = z * 0.5 * (1.0 + lax.erf(z * 0.7071067811865476))
    mu = jnp.mean(a, axis=-1, keepdims=True)
    var = jnp.mean((a - mu) ** 2, axis=-1, keepdims=True)
    he = (a - mu) * lax.rsqrt(var + 1e-5) * g_ref[...] + beta_ref[...]
    out_ref[...] = jnp.dot(he, w2t_ref[...],
                           preferred_element_type=jnp.float32)


def _edge_mlp(he_parts, w1t, b1, g1, beta1, w2t):
    blk = 512
    grid = (E // blk,)
    return pl.pallas_call(
        _edge_mlp_body,
        grid=grid,
        in_specs=[
            pl.BlockSpec((NC, blk, D), lambda i: (0, i, 0)),
            pl.BlockSpec((D, D), lambda i: (0, 0)),
            pl.BlockSpec((1, D), lambda i: (0, 0)),
            pl.BlockSpec((1, D), lambda i: (0, 0)),
            pl.BlockSpec((1, D), lambda i: (0, 0)),
            pl.BlockSpec((D, D), lambda i: (0, 0)),
        ],
        out_specs=pl.BlockSpec((blk, D), lambda i: (i, 0)),
        out_shape=jax.ShapeDtypeStruct((E, D), jnp.float32),
    )(he_parts, w1t, b1, g1, beta1, w2t)


def kernel(X, edge_idx, edge_w, W1, b1, g1, beta1, W2, b2, g2, beta2):
    x = X[0]
    idx = edge_idx[0]
    w = edge_w[0]

    pad = NPAD - N
    xp = jnp.pad(x, ((0, pad), (0, 0)))
    idxp = jnp.pad(idx, ((0, pad), (0, 0)))
    wp = jnp.pad(w, ((0, pad), (0, 0)))

    x_t = xp.reshape(NT, NPT, D)
    idx_t = idxp.reshape(NT, NCHUNK, ROWS)
    w_t = wp.reshape(NT, NPT, K)
    zeros = jnp.zeros((E, D), jnp.float32)
    prm = jnp.stack([b2, g2, beta2])

    he_parts = _scatter_kernel(x_t, idx_t, w_t, zeros)
    g_tab = _edge_mlp(he_parts, W1.T, b1.reshape(1, D), g1.reshape(1, D),
                      beta1.reshape(1, D), W2.T)
    out_t = _gather_kernel(g_tab, idx_t, w_t, x_t, prm)
    return out_t.reshape(NPAD, D)[:N].reshape(1, N, D)


# speedup vs baseline: 1.1931x; 1.1931x over previous
import functools

import jax
import jax.numpy as jnp
from jax import lax
from jax.experimental import pallas as pl
from jax.experimental.pallas import tpu as pltpu
from jax.experimental.pallas import tpu_sc as plsc

N, K, D, E = 10000, 32, 128, 4096
NC, NS = 2, 16
NT = NC * NS
NPT = 324
NPAD = NT * NPT
SCN = 4
SROWS = SCN * K
SNCHUNK = NPT // SCN
GCN = 2
GROWS = GCN * K
GNCHUNK = NPT // GCN
EPT = E // NS

_mesh = plsc.VectorSubcoreMesh(core_axis_name="c", subcore_axis_name="s")


@functools.partial(
    pl.kernel,
    mesh=_mesh,
    out_type=jax.ShapeDtypeStruct((NC, E, D), jnp.float32),
    scratch_types=[
        pltpu.VMEM((2 * SCN, D), jnp.float32),
        pltpu.VMEM((SNCHUNK, SROWS), jnp.int32),
        pltpu.VMEM((NPT, K), jnp.float32),
        pltpu.VMEM((2 * SROWS, D), jnp.float32),
        pltpu.VMEM_SHARED((E, D), jnp.float32),
        pltpu.SemaphoreType.DMA,
        pltpu.SemaphoreType.DMA,
    ],
)
def _scatter_kernel(x_hbm, idx_hbm, w_hbm, z_hbm, out_hbm,
                    xc_v, idx_v, w_v, contrib_v, he_sh, semx, sems):
    c = lax.axis_index("c")
    s = lax.axis_index("s")
    t = c * NS + s
    pltpu.sync_copy(idx_hbm.at[t], idx_v)
    pltpu.sync_copy(w_hbm.at[t], w_v)
    pltpu.sync_copy(z_hbm.at[pl.ds(s * EPT, EPT)],
                    he_sh.at[pl.ds(s * EPT, EPT)])
    plsc.subcore_barrier()

    def x_copy(j, p):
        return pltpu.make_async_copy(
            x_hbm.at[t, pl.ds(j * SCN, SCN)],
            xc_v.at[pl.ds(p * SCN, SCN)], semx)

    def scat_copy(j, p):
        return pltpu.make_async_copy(
            contrib_v.at[pl.ds(p * SROWS, SROWS)],
            he_sh.at[idx_v.at[j]], sems)

    def compute(j, p):
        for i in range(SCN):
            node = j * SCN + i
            wvecs = [w_v[node, pl.ds(h * 16, 16)] for h in range(K // 16)]
            ws = [wvecs[k // 16][k % 16] for k in range(K)]
            for col in range(D // 16):
                vx = xc_v[p * SCN + i, pl.ds(col * 16, 16)]
                for k in range(K):
                    contrib_v[p * SROWS + i * K + k,
                              pl.ds(col * 16, 16)] = vx * ws[k]

    def step(j, p):
        x_copy(j, p).wait()
        x_copy(jnp.minimum(j + 1, SNCHUNK - 1), 1 - p).start()
        compute(j, p)
        scat_copy(j - 1, 1 - p).wait()
        scat_copy(j, p).start(add=True)

    x_copy(0, 0).start()
    x_copy(0, 0).wait()
    x_copy(1, 1).start()
    compute(0, 0)
    scat_copy(0, 0).start(add=True)

    def pair(m, carry):
        step(2 * m + 1, 1)
        step(2 * m + 2, 0)
        return carry

    lax.fori_loop(0, (SNCHUNK - 1) // 2, pair, 0)
    scat_copy(SNCHUNK - 1, 0).wait()
    x_copy(SNCHUNK - 1, 1).wait()
    plsc.subcore_barrier()
    pltpu.sync_copy(he_sh.at[pl.ds(s * EPT, EPT)],
                    out_hbm.at[c, pl.ds(s * EPT, EPT)])


@functools.partial(
    pl.kernel,
    mesh=_mesh,
    out_type=jax.ShapeDtypeStruct((NT, NPT, D), jnp.float32),
    scratch_types=[
        pltpu.VMEM((GNCHUNK, GROWS), jnp.int32),
        pltpu.VMEM((NPT, K), jnp.float32),
        pltpu.VMEM((2 * GROWS, D), jnp.float32),
        pltpu.VMEM((2 * GCN, D), jnp.float32),
        pltpu.VMEM_SHARED((E, D), jnp.float32),
        pltpu.SemaphoreType.DMA,
        pltpu.SemaphoreType.DMA,
    ],
)
def _gather_kernel(he_hbm, idx_hbm, w_hbm, out_hbm,
                   idx_v, w_v, rows_v, xnc_v, he_sh, semg, semo):
    c = lax.axis_index("c")
    s = lax.axis_index("s")
    t = c * NS + s
    pltpu.sync_copy(idx_hbm.at[t], idx_v)
    pltpu.sync_copy(w_hbm.at[t], w_v)
    pltpu.sync_copy(he_hbm.at[pl.ds(s * EPT, EPT)],
                    he_sh.at[pl.ds(s * EPT, EPT)])
    plsc.subcore_barrier()

    def g_copy(j, p):
        return pltpu.make_async_copy(
            he_sh.at[idx_v.at[j]], rows_v.at[pl.ds(p * GROWS, GROWS)], semg)

    def o_copy(j, p):
        return pltpu.make_async_copy(
            xnc_v.at[pl.ds(p * GCN, GCN)],
            out_hbm.at[t, pl.ds(j * GCN, GCN)], semo)

    def compute(j, p):
        for i in range(GCN):
            node = j * GCN + i
            wvecs = [w_v[node, pl.ds(h * 16, 16)] for h in range(K // 16)]
            ws = [wvecs[k // 16][k % 16] for k in range(K)]
            base = p * GROWS + i * K
            for col in range(D // 16):
                acc = rows_v[base, pl.ds(col * 16, 16)] * ws[0]
                for k in range(1, K):
                    acc = acc + rows_v[base + k, pl.ds(col * 16, 16)] * ws[k]
                xnc_v[p * GCN + i, pl.ds(col * 16, 16)] = acc

    def step(j, p):
        g_copy(j, p).wait()
        g_copy(jnp.minimum(j + 1, GNCHUNK - 1), 1 - p).start()
        compute(j, p)
        o_copy(j - 1, 1 - p).wait()
        o_copy(j, p).start()

    g_copy(0, 0).start()
    g_copy(0, 0).wait()
    g_copy(1, 1).start()
    compute(0, 0)
    o_copy(0, 0).start()

    def pair(m, carry):
        step(2 * m + 1, 1)
        step(2 * m + 2, 0)
        return carry

    lax.fori_loop(0, (GNCHUNK - 2) // 2, pair, 0)
    step(GNCHUNK - 1, 1)
    o_copy(GNCHUNK - 1, 1).wait()
    g_copy(GNCHUNK - 1, 0).wait()


def _mlp_ln_body(h, w_ref, b_ref, g_ref, beta_ref):
    z = jnp.dot(h, w_ref[...], preferred_element_type=jnp.float32)
    z = z + b_ref[...]
    a = z * 0.5 * (1.0 + lax.erf(z * 0.7071067811865476))
    mu = jnp.mean(a, axis=-1, keepdims=True)
    var = jnp.mean((a - mu) ** 2, axis=-1, keepdims=True)
    return (a - mu) * lax.rsqrt(var + 1e-5) * g_ref[...] + beta_ref[...]


def _edge_mlp_body(hp_ref, w_ref, b_ref, g_ref, beta_ref, out_ref):
    h = hp_ref[0] + hp_ref[1]
    out_ref[...] = _mlp_ln_body(h, w_ref, b_ref, g_ref, beta_ref)


def _node_mlp_body(xn_ref, x_ref, w_ref, b_ref, g_ref, beta_ref, out_ref):
    y = _mlp_ln_body(xn_ref[...], w_ref, b_ref, g_ref, beta_ref)
    out_ref[...] = y + x_ref[...]


def _edge_mlp(he_parts, w1t, b1, g1, beta1):
    blk = 512
    grid = (E // blk,)
    return pl.pallas_call(
        _edge_mlp_body,
        grid=grid,
        in_specs=[
            pl.BlockSpec((NC, blk, D), lambda i: (0, i, 0)),
            pl.BlockSpec((D, D), lambda i: (0, 0)),
            pl.BlockSpec((1, D), lambda i: (0, 0)),
            pl.BlockSpec((1, D), lambda i: (0, 0)),
            pl.BlockSpec((1, D), lambda i: (0, 0)),
        ],
        out_specs=pl.BlockSpec((blk, D), lambda i: (i, 0)),
        out_shape=jax.ShapeDtypeStruct((E, D), jnp.float32),
    )(he_parts, w1t, b1, g1, beta1)


def _node_mlp(xn, x, w2t, b2, g2, beta2):
    blk = 1296
    grid = (NPAD // blk,)
    return pl.pallas_call(
        _node_mlp_body,
        grid=grid,
        in_specs=[
            pl.BlockSpec((blk, D), lambda i: (i, 0)),
            pl.BlockSpec((blk, D), lambda i: (i, 0)),
            pl.BlockSpec((D, D), lambda i: (0, 0)),
            pl.BlockSpec((1, D), lambda i: (0, 0)),
            pl.BlockSpec((1, D), lambda i: (0, 0)),
            pl.BlockSpec((1, D), lambda i: (0, 0)),
        ],
        out_specs=pl.BlockSpec((blk, D), lambda i: (i, 0)),
        out_shape=jax.ShapeDtypeStruct((NPAD, D), jnp.float32),
    )(xn, x, w2t, b2, g2, beta2)


def kernel(X, edge_idx, edge_w, W1, b1, g1, beta1, W2, b2, g2, beta2):
    x = X[0]
    idx = edge_idx[0]
    w = edge_w[0]

    pad = NPAD - N
    xp = jnp.pad(x, ((0, pad), (0, 0)))
    idxp = jnp.pad(idx, ((0, pad), (0, 0)))
    wp = jnp.pad(w, ((0, pad), (0, 0)))

    x_t = xp.reshape(NT, NPT, D)
    idx_s = idxp.reshape(NT, SNCHUNK, SROWS)
    idx_g = idxp.reshape(NT, GNCHUNK, GROWS)
    w_t = wp.reshape(NT, NPT, K)
    zeros = jnp.zeros((E, D), jnp.float32)

    he_parts = _scatter_kernel(x_t, idx_s, w_t, zeros)
    he = _edge_mlp(he_parts, W1.T, b1.reshape(1, D), g1.reshape(1, D),
                   beta1.reshape(1, D))
    xn_t = _gather_kernel(he, idx_g, w_t)
    out = _node_mlp(xn_t.reshape(NPAD, D), xp, W2.T, b2.reshape(1, D),
                    g2.reshape(1, D), beta2.reshape(1, D))
    return out[:N].reshape(1, N, D)


# speedup vs baseline: 1.4359x; 1.2035x over previous
import functools

import jax
import jax.numpy as jnp
from jax import lax
from jax.experimental import pallas as pl
from jax.experimental.pallas import tpu as pltpu
from jax.experimental.pallas import tpu_sc as plsc

N, K, D, E = 10000, 32, 128, 4096
NC, NS = 2, 16
NT = NC * NS
NPT = 320
NPAD = NT * NPT
SCN = 2
SROWS = SCN * K
SNCHUNK = NPT // SCN
GCN = 2
GROWS = GCN * K
GNCHUNK = NPT // GCN
EPT = E // NS

_mesh = plsc.VectorSubcoreMesh(core_axis_name="c", subcore_axis_name="s")


@functools.partial(
    pl.kernel,
    mesh=_mesh,
    out_type=jax.ShapeDtypeStruct((NC, E, D), jnp.float32),
    scratch_types=[
        pltpu.VMEM((2 * SCN, D), jnp.float32),
        pltpu.VMEM((SNCHUNK, SROWS), jnp.int32),
        pltpu.VMEM((NPT, K), jnp.float32),
        pltpu.VMEM((2 * SROWS, D), jnp.float32),
        pltpu.VMEM_SHARED((E, D), jnp.float32),
        pltpu.SemaphoreType.DMA,
        pltpu.SemaphoreType.DMA,
    ],
)
def _scatter_kernel(x_hbm, idx_hbm, w_hbm, z_hbm, out_hbm,
                    xc_v, idx_v, w_v, contrib_v, he_sh, semx, sems):
    c = lax.axis_index("c")
    s = lax.axis_index("s")
    t = c * NS + s
    pltpu.sync_copy(idx_hbm.at[t], idx_v)
    pltpu.sync_copy(w_hbm.at[t], w_v)
    pltpu.sync_copy(z_hbm.at[pl.ds(s * EPT, EPT)],
                    he_sh.at[pl.ds(s * EPT, EPT)])
    plsc.subcore_barrier()

    def x_copy(j, p):
        return pltpu.make_async_copy(
            x_hbm.at[t, pl.ds(j * SCN, SCN)],
            xc_v.at[pl.ds(p * SCN, SCN)], semx)

    def scat_copy(j, p):
        return pltpu.make_async_copy(
            contrib_v.at[pl.ds(p * SROWS, SROWS)],
            he_sh.at[idx_v.at[j]], sems)

    def compute(j, p):
        for i in range(SCN):
            node = j * SCN + i
            wvecs = [w_v[node, pl.ds(h * 16, 16)] for h in range(K // 16)]
            ws = [wvecs[k // 16][k % 16] for k in range(K)]
            for col in range(D // 16):
                vx = xc_v[p * SCN + i, pl.ds(col * 16, 16)]
                for k in range(K):
                    contrib_v[p * SROWS + i * K + k,
                              pl.ds(col * 16, 16)] = vx * ws[k]

    def step(j, p):
        x_copy(j, p).wait()
        x_copy(jnp.minimum(j + 1, SNCHUNK - 1), 1 - p).start()
        compute(j, p)
        scat_copy(j - 1, 1 - p).wait()
        scat_copy(j, p).start(add=True)

    x_copy(0, 0).start()
    x_copy(0, 0).wait()
    x_copy(1, 1).start()
    compute(0, 0)
    scat_copy(0, 0).start(add=True)

    def pair(m, carry):
        step(2 * m + 1, 1)
        step(2 * m + 2, 0)
        return carry

    lax.fori_loop(0, (SNCHUNK - 2) // 2, pair, 0)
    step(SNCHUNK - 1, 1)
    scat_copy(SNCHUNK - 1, 1).wait()
    x_copy(SNCHUNK - 1, 0).wait()
    plsc.subcore_barrier()
    pltpu.sync_copy(he_sh.at[pl.ds(s * EPT, EPT)],
                    out_hbm.at[c, pl.ds(s * EPT, EPT)])


@functools.partial(
    pl.kernel,
    mesh=_mesh,
    out_type=jax.ShapeDtypeStruct((NT, NPT, D), jnp.float32),
    scratch_types=[
        pltpu.VMEM((GNCHUNK, GROWS), jnp.int32),
        pltpu.VMEM((NPT, K), jnp.float32),
        pltpu.VMEM((2 * GROWS, D), jnp.float32),
        pltpu.VMEM((2 * GCN, D), jnp.float32),
        pltpu.VMEM_SHARED((E, D), jnp.float32),
        pltpu.SemaphoreType.DMA,
        pltpu.SemaphoreType.DMA,
    ],
)
def _gather_kernel(he_hbm, idx_hbm, w_hbm, out_hbm,
                   idx_v, w_v, rows_v, xnc_v, he_sh, semg, semo):
    c = lax.axis_index("c")
    s = lax.axis_index("s")
    t = c * NS + s
    pltpu.sync_copy(idx_hbm.at[t], idx_v)
    pltpu.sync_copy(w_hbm.at[t], w_v)
    pltpu.sync_copy(he_hbm.at[pl.ds(s * EPT, EPT)],
                    he_sh.at[pl.ds(s * EPT, EPT)])
    plsc.subcore_barrier()

    def g_copy(j, p):
        return pltpu.make_async_copy(
            he_sh.at[idx_v.at[j]], rows_v.at[pl.ds(p * GROWS, GROWS)], semg)

    def o_copy(j, p):
        return pltpu.make_async_copy(
            xnc_v.at[pl.ds(p * GCN, GCN)],
            out_hbm.at[t, pl.ds(j * GCN, GCN)], semo)

    def compute(j, p):
        for i in range(GCN):
            node = j * GCN + i
            wvecs = [w_v[node, pl.ds(h * 16, 16)] for h in range(K // 16)]
            ws = [wvecs[k // 16][k % 16] for k in range(K)]
            base = p * GROWS + i * K
            for col in range(D // 16):
                acc = rows_v[base, pl.ds(col * 16, 16)] * ws[0]
                for k in range(1, K):
                    acc = acc + rows_v[base + k, pl.ds(col * 16, 16)] * ws[k]
                xnc_v[p * GCN + i, pl.ds(col * 16, 16)] = acc

    def step(j, p):
        g_copy(j, p).wait()
        g_copy(jnp.minimum(j + 1, GNCHUNK - 1), 1 - p).start()
        compute(j, p)
        o_copy(j - 1, 1 - p).wait()
        o_copy(j, p).start()

    g_copy(0, 0).start()
    g_copy(0, 0).wait()
    g_copy(1, 1).start()
    compute(0, 0)
    o_copy(0, 0).start()

    def pair(m, carry):
        step(2 * m + 1, 1)
        step(2 * m + 2, 0)
        return carry

    lax.fori_loop(0, (GNCHUNK - 2) // 2, pair, 0)
    step(GNCHUNK - 1, 1)
    o_copy(GNCHUNK - 1, 1).wait()
    g_copy(GNCHUNK - 1, 0).wait()


def _mlp_ln_body(h, w_ref, b_ref, g_ref, beta_ref):
    z = jnp.dot(h, w_ref[...], preferred_element_type=jnp.float32)
    z = z + b_ref[...]
    a = z * 0.5 * (1.0 + lax.erf(z * 0.7071067811865476))
    mu = jnp.mean(a, axis=-1, keepdims=True)
    var = jnp.mean((a - mu) ** 2, axis=-1, keepdims=True)
    return (a - mu) * lax.rsqrt(var + 1e-5) * g_ref[...] + beta_ref[...]


def _edge_mlp_body(hp_ref, w_ref, b_ref, g_ref, beta_ref, out_ref):
    h = hp_ref[0] + hp_ref[1]
    out_ref[...] = _mlp_ln_body(h, w_ref, b_ref, g_ref, beta_ref)


def _node_mlp_body(xn_ref, x_ref, w_ref, b_ref, g_ref, beta_ref, out_ref):
    y = _mlp_ln_body(xn_ref[...], w_ref, b_ref, g_ref, beta_ref)
    out_ref[...] = y + x_ref[...]


def _edge_mlp(he_parts, w1t, b1, g1, beta1):
    blk = 512
    grid = (E // blk,)
    return pl.pallas_call(
        _edge_mlp_body,
        grid=grid,
        in_specs=[
            pl.BlockSpec((NC, blk, D), lambda i: (0, i, 0)),
            pl.BlockSpec((D, D), lambda i: (0, 0)),
            pl.BlockSpec((1, D), lambda i: (0, 0)),
            pl.BlockSpec((1, D), lambda i: (0, 0)),
            pl.BlockSpec((1, D), lambda i: (0, 0)),
        ],
        out_specs=pl.BlockSpec((blk, D), lambda i: (i, 0)),
        out_shape=jax.ShapeDtypeStruct((E, D), jnp.float32),
    )(he_parts, w1t, b1, g1, beta1)


def _node_mlp(xn, x, w2t, b2, g2, beta2):
    blk = 1024
    grid = (NPAD // blk,)
    return pl.pallas_call(
        _node_mlp_body,
        grid=grid,
        in_specs=[
            pl.BlockSpec((blk, D), lambda i: (i, 0)),
            pl.BlockSpec((blk, D), lambda i: (i, 0)),
            pl.BlockSpec((D, D), lambda i: (0, 0)),
            pl.BlockSpec((1, D), lambda i: (0, 0)),
            pl.BlockSpec((1, D), lambda i: (0, 0)),
            pl.BlockSpec((1, D), lambda i: (0, 0)),
        ],
        out_specs=pl.BlockSpec((blk, D), lambda i: (i, 0)),
        out_shape=jax.ShapeDtypeStruct((NPAD, D), jnp.float32),
    )(xn, x, w2t, b2, g2, beta2)


def kernel(X, edge_idx, edge_w, W1, b1, g1, beta1, W2, b2, g2, beta2):
    x = X[0]
    idx = edge_idx[0]
    w = edge_w[0]

    pad = NPAD - N
    xp = jnp.pad(x, ((0, pad), (0, 0)))
    idxp = jnp.pad(idx, ((0, pad), (0, 0)))
    wp = jnp.pad(w, ((0, pad), (0, 0)))

    x_t = xp.reshape(NT, NPT, D)
    idx_s = idxp.reshape(NT, SNCHUNK, SROWS)
    idx_g = idxp.reshape(NT, GNCHUNK, GROWS)
    w_t = wp.reshape(NT, NPT, K)
    zeros = jnp.zeros((E, D), jnp.float32)

    he_parts = _scatter_kernel(x_t, idx_s, w_t, zeros)
    he = _edge_mlp(he_parts, W1.T, b1.reshape(1, D), g1.reshape(1, D),
                   beta1.reshape(1, D))
    xn_t = _gather_kernel(he, idx_g, w_t)
    out = _node_mlp(xn_t.reshape(NPAD, D), xp, W2.T, b2.reshape(1, D),
                    g2.reshape(1, D), beta2.reshape(1, D))
    return out[:N].reshape(1, N, D)


# speedup vs baseline: 1.4669x; 1.0216x over previous
import functools

import jax
import jax.numpy as jnp
from jax import lax
from jax.experimental import pallas as pl
from jax.experimental.pallas import tpu as pltpu
from jax.experimental.pallas import tpu_sc as plsc

N, K, D, E = 10000, 32, 128, 4096
NC, NS = 2, 16
NT = NC * NS
NPT = 320
NPAD = NT * NPT
SCN = 2
SROWS = SCN * K
SNCHUNK = NPT // SCN
GCN = 2
GROWS = GCN * K
GNCHUNK = NPT // GCN
EPT = E // NS

_mesh = plsc.VectorSubcoreMesh(core_axis_name="c", subcore_axis_name="s")


@functools.partial(
    pl.kernel,
    mesh=_mesh,
    out_type=jax.ShapeDtypeStruct((NC, E, D), jnp.float32),
    scratch_types=[
        pltpu.VMEM((2 * SCN, D), jnp.float32),
        pltpu.VMEM((SNCHUNK, SROWS), jnp.int32),
        pltpu.VMEM((NPT, K), jnp.float32),
        pltpu.VMEM((2 * SROWS, D), jnp.float32),
        pltpu.VMEM_SHARED((E, D), jnp.float32),
        pltpu.SemaphoreType.DMA,
        pltpu.SemaphoreType.DMA,
    ],
)
def _scatter_kernel(x_hbm, idx_hbm, w_hbm, z_hbm, out_hbm,
                    xc_v, idx_v, w_v, contrib_v, he_sh, semx, sems):
    c = lax.axis_index("c")
    s = lax.axis_index("s")
    t = c * NS + s
    pltpu.async_copy(idx_hbm.at[t], idx_v, sems)
    pltpu.async_copy(w_hbm.at[t], w_v, sems)
    pltpu.async_copy(z_hbm.at[pl.ds(s * EPT, EPT)],
                     he_sh.at[pl.ds(s * EPT, EPT)], sems)
    pltpu.make_async_copy(idx_hbm.at[t], idx_v, sems).wait()
    pltpu.make_async_copy(w_hbm.at[t], w_v, sems).wait()
    pltpu.make_async_copy(z_hbm.at[pl.ds(s * EPT, EPT)],
                          he_sh.at[pl.ds(s * EPT, EPT)], sems).wait()
    plsc.subcore_barrier()

    def x_copy(j, p):
        return pltpu.make_async_copy(
            x_hbm.at[t, pl.ds(j * SCN, SCN)],
            xc_v.at[pl.ds(p * SCN, SCN)], semx)

    def scat_copy(j, p):
        return pltpu.make_async_copy(
            contrib_v.at[pl.ds(p * SROWS, SROWS)],
            he_sh.at[idx_v.at[j]], sems)

    def compute(j, p):
        for i in range(SCN):
            node = j * SCN + i
            wvecs = [w_v[node, pl.ds(h * 16, 16)] for h in range(K // 16)]
            ws = [wvecs[k // 16][k % 16] for k in range(K)]
            for col in range(D // 16):
                vx = xc_v[p * SCN + i, pl.ds(col * 16, 16)]
                for k in range(K):
                    contrib_v[p * SROWS + i * K + k,
                              pl.ds(col * 16, 16)] = vx * ws[k]

    def step(j, p):
        x_copy(j, p).wait()
        x_copy(jnp.minimum(j + 1, SNCHUNK - 1), 1 - p).start()
        compute(j, p)
        scat_copy(j - 1, 1 - p).wait()
        scat_copy(j, p).start(add=True)

    x_copy(0, 0).start()
    x_copy(0, 0).wait()
    x_copy(1, 1).start()
    compute(0, 0)
    scat_copy(0, 0).start(add=True)

    def pair(m, carry):
        step(2 * m + 1, 1)
        step(2 * m + 2, 0)
        return carry

    lax.fori_loop(0, (SNCHUNK - 2) // 2, pair, 0)
    step(SNCHUNK - 1, 1)
    scat_copy(SNCHUNK - 1, 1).wait()
    x_copy(SNCHUNK - 1, 0).wait()
    plsc.subcore_barrier()
    pltpu.sync_copy(he_sh.at[pl.ds(s * EPT, EPT)],
                    out_hbm.at[c, pl.ds(s * EPT, EPT)])


@functools.partial(
    pl.kernel,
    mesh=_mesh,
    out_type=jax.ShapeDtypeStruct((NT, NPT, D), jnp.float32),
    scratch_types=[
        pltpu.VMEM((GNCHUNK, GROWS), jnp.int32),
        pltpu.VMEM((NPT, K), jnp.float32),
        pltpu.VMEM((2 * GROWS, D), jnp.float32),
        pltpu.VMEM((2 * GCN, D), jnp.float32),
        pltpu.VMEM_SHARED((E, D), jnp.float32),
        pltpu.SemaphoreType.DMA,
        pltpu.SemaphoreType.DMA,
    ],
)
def _gather_kernel(he_hbm, idx_hbm, w_hbm, out_hbm,
                   idx_v, w_v, rows_v, xnc_v, he_sh, semg, semo):
    c = lax.axis_index("c")
    s = lax.axis_index("s")
    t = c * NS + s
    pltpu.async_copy(idx_hbm.at[t], idx_v, semg)
    pltpu.async_copy(w_hbm.at[t], w_v, semg)
    pltpu.async_copy(he_hbm.at[pl.ds(s * EPT, EPT)],
                     he_sh.at[pl.ds(s * EPT, EPT)], semg)
    pltpu.make_async_copy(idx_hbm.at[t], idx_v, semg).wait()
    pltpu.make_async_copy(w_hbm.at[t], w_v, semg).wait()
    pltpu.make_async_copy(he_hbm.at[pl.ds(s * EPT, EPT)],
                          he_sh.at[pl.ds(s * EPT, EPT)], semg).wait()
    plsc.subcore_barrier()

    def g_copy(j, p):
        return pltpu.make_async_copy(
            he_sh.at[idx_v.at[j]], rows_v.at[pl.ds(p * GROWS, GROWS)], semg)

    def o_copy(j, p):
        return pltpu.make_async_copy(
            xnc_v.at[pl.ds(p * GCN, GCN)],
            out_hbm.at[t, pl.ds(j * GCN, GCN)], semo)

    def compute(j, p):
        for i in range(GCN):
            node = j * GCN + i
            wvecs = [w_v[node, pl.ds(h * 16, 16)] for h in range(K // 16)]
            ws = [wvecs[k // 16][k % 16] for k in range(K)]
            base = p * GROWS + i * K
            for col in range(D // 16):
                acc = rows_v[base, pl.ds(col * 16, 16)] * ws[0]
                for k in range(1, K):
                    acc = acc + rows_v[base + k, pl.ds(col * 16, 16)] * ws[k]
                xnc_v[p * GCN + i, pl.ds(col * 16, 16)] = acc

    def step(j, p):
        g_copy(j, p).wait()
        g_copy(jnp.minimum(j + 1, GNCHUNK - 1), 1 - p).start()
        compute(j, p)
        o_copy(j - 1, 1 - p).wait()
        o_copy(j, p).start()

    g_copy(0, 0).start()
    g_copy(0, 0).wait()
    g_copy(1, 1).start()
    compute(0, 0)
    o_copy(0, 0).start()

    def pair(m, carry):
        step(2 * m + 1, 1)
        step(2 * m + 2, 0)
        return carry

    lax.fori_loop(0, (GNCHUNK - 2) // 2, pair, 0)
    step(GNCHUNK - 1, 1)
    o_copy(GNCHUNK - 1, 1).wait()
    g_copy(GNCHUNK - 1, 0).wait()


def _mlp_ln_body(h, w_ref, b_ref, g_ref, beta_ref):
    z = jnp.dot(h, w_ref[...], preferred_element_type=jnp.float32)
    z = z + b_ref[...]
    a = z * 0.5 * (1.0 + lax.erf(z * 0.7071067811865476))
    mu = jnp.mean(a, axis=-1, keepdims=True)
    var = jnp.mean((a - mu) ** 2, axis=-1, keepdims=True)
    return (a - mu) * lax.rsqrt(var + 1e-5) * g_ref[...] + beta_ref[...]


def _edge_mlp_body(hp_ref, w_ref, b_ref, g_ref, beta_ref, out_ref):
    h = hp_ref[0] + hp_ref[1]
    out_ref[...] = _mlp_ln_body(h, w_ref, b_ref, g_ref, beta_ref)


def _node_mlp_body(xn_ref, x_ref, w_ref, b_ref, g_ref, beta_ref, out_ref):
    y = _mlp_ln_body(xn_ref[...], w_ref, b_ref, g_ref, beta_ref)
    out_ref[...] = y + x_ref[...]


def _edge_mlp(he_parts, w1t, b1, g1, beta1):
    blk = 512
    grid = (E // blk,)
    return pl.pallas_call(
        _edge_mlp_body,
        grid=grid,
        in_specs=[
            pl.BlockSpec((NC, blk, D), lambda i: (0, i, 0)),
            pl.BlockSpec((D, D), lambda i: (0, 0)),
            pl.BlockSpec((1, D), lambda i: (0, 0)),
            pl.BlockSpec((1, D), lambda i: (0, 0)),
            pl.BlockSpec((1, D), lambda i: (0, 0)),
        ],
        out_specs=pl.BlockSpec((blk, D), lambda i: (i, 0)),
        out_shape=jax.ShapeDtypeStruct((E, D), jnp.float32),
    )(he_parts, w1t, b1, g1, beta1)


def _node_mlp(xn, x, w2t, b2, g2, beta2):
    blk = 1024
    grid = (NPAD // blk,)
    return pl.pallas_call(
        _node_mlp_body,
        grid=grid,
        in_specs=[
            pl.BlockSpec((blk, D), lambda i: (i, 0)),
            pl.BlockSpec((blk, D), lambda i: (i, 0)),
            pl.BlockSpec((D, D), lambda i: (0, 0)),
            pl.BlockSpec((1, D), lambda i: (0, 0)),
            pl.BlockSpec((1, D), lambda i: (0, 0)),
            pl.BlockSpec((1, D), lambda i: (0, 0)),
        ],
        out_specs=pl.BlockSpec((blk, D), lambda i: (i, 0)),
        out_shape=jax.ShapeDtypeStruct((NPAD, D), jnp.float32),
    )(xn, x, w2t, b2, g2, beta2)


def kernel(X, edge_idx, edge_w, W1, b1, g1, beta1, W2, b2, g2, beta2):
    x = X[0]
    idx = edge_idx[0]
    w = edge_w[0]

    pad = NPAD - N
    xp = jnp.pad(x, ((0, pad), (0, 0)))
    idxp = jnp.pad(idx, ((0, pad), (0, 0)))
    wp = jnp.pad(w, ((0, pad), (0, 0)))

    x_t = xp.reshape(NT, NPT, D)
    idx_s = idxp.reshape(NT, SNCHUNK, SROWS)
    idx_g = idxp.reshape(NT, GNCHUNK, GROWS)
    w_t = wp.reshape(NT, NPT, K)
    zeros = jnp.zeros((E, D), jnp.float32)

    he_parts = _scatter_kernel(x_t, idx_s, w_t, zeros)
    he = _edge_mlp(he_parts, W1.T, b1.reshape(1, D), g1.reshape(1, D),
                   beta1.reshape(1, D))
    xn_t = _gather_kernel(he, idx_g, w_t)
    out = _node_mlp(xn_t.reshape(NPAD, D), xp, W2.T, b2.reshape(1, D),
                    g2.reshape(1, D), beta2.reshape(1, D))
    return out[:N].reshape(1, N, D)
